# baseline (speedup 1.0000x reference)
import functools

import jax
import jax.numpy as jnp
from jax import lax
from jax.experimental import pallas as pl
from jax.experimental.pallas import tpu as pltpu
from jax.experimental.pallas import tpu_sc as plsc

N_ROWS = 100000
D = 128
NC = 2
NS = 16
NW = NC * NS

CHUNK = 64
RW = 3128
RW31 = 3032
NFULL = 48
TAIL = RW - NFULL * CHUNK
TAIL31 = RW31 - 47 * CHUNK
NBUF = 12

_mesh = plsc.VectorSubcoreMesh(core_axis_name="c", subcore_axis_name="s")

_scratch = (
    [pltpu.VMEM((RW,), jnp.int32)]
    + [pltpu.VMEM((CHUNK, D), jnp.float32) for _ in range(NBUF)]
    + [pltpu.VMEM((TAIL, D), jnp.float32)]
    + [pltpu.SemaphoreType.DMA for _ in range(2 * NBUF + 2)]
)


@functools.partial(
    pl.kernel,
    out_type=jax.ShapeDtypeStruct((N_ROWS, D), jnp.float32),
    mesh=_mesh,
    scratch_types=_scratch,
)
def _gather_kernel(idx_hbm, tbl_hbm, out_hbm, idx_v, *rest):
    bufs = rest[:NBUF]
    tbuf = rest[NBUF]
    gsems = rest[NBUF + 1:2 * NBUF + 1]
    ssems = rest[2 * NBUF + 1:3 * NBUF + 1]
    tgsem = rest[3 * NBUF + 1]
    tssem = rest[3 * NBUF + 2]

    w = lax.axis_index("s") * NC + lax.axis_index("c")
    r0 = w * RW

    @pl.when(w < NW - 1)
    def _():
        pltpu.sync_copy(idx_hbm.at[pl.ds(r0, RW)], idx_v)

    @pl.when(w == NW - 1)
    def _():
        pltpu.sync_copy(idx_hbm.at[pl.ds(r0, RW31)], idx_v.at[pl.ds(0, RW31)])

    def start_gather(c, b):
        pltpu.async_copy(
            tbl_hbm.at[idx_v.at[pl.ds(c * CHUNK, CHUNK)]], bufs[b], gsems[b]
        )

    def start_store(c, b):
        pltpu.async_copy(
            bufs[b], out_hbm.at[pl.ds(r0 + c * CHUNK, CHUNK)], ssems[b]
        )

    def wait_gather(b):
        pltpu.make_async_copy(
            tbl_hbm.at[idx_v.at[pl.ds(0, CHUNK)]], bufs[b], gsems[b]
        ).wait()

    def wait_store(b):
        pltpu.make_async_copy(
            bufs[b], out_hbm.at[pl.ds(0, CHUNK)], ssems[b]
        ).wait()

    for c in range(NBUF - 1):
        start_gather(c, c)

    @pl.when(w < NW - 1)
    def _():
        pltpu.async_copy(
            tbl_hbm.at[idx_v.at[pl.ds(NFULL * CHUNK, TAIL)]], tbuf, tgsem
        )

    @pl.when(w == NW - 1)
    def _():
        pltpu.async_copy(
            tbl_hbm.at[idx_v.at[pl.ds(47 * CHUNK, TAIL31)]],
            tbuf.at[pl.ds(0, TAIL31)],
            tgsem,
        )

    for c in range(NBUF):
        wait_gather(c)
        start_store(c, c)
        if c >= 1:
            wait_store(c - 1)
        start_gather(c + NBUF - 1, (c + NBUF - 1) % NBUF)

    @pl.loop(1, 3)
    def _(i):
        c0 = i * NBUF
        for b in range(NBUF):
            wait_gather(b)
            start_store(c0 + b, b)
            wait_store((b + NBUF - 1) % NBUF)
            start_gather(c0 + b + NBUF - 1, (b + NBUF - 1) % NBUF)

    wait_gather(0)
    start_store(36, 0)
    wait_store(NBUF - 1)

    @pl.when(w < NW - 1)
    def _():
        start_gather(47, NBUF - 1)

    for c in range(37, 47):
        wait_gather(c % NBUF)
        start_store(c, c % NBUF)

    @pl.when(w < NW - 1)
    def _():
        wait_gather(NBUF - 1)
        start_store(47, NBUF - 1)

    for b in range(NBUF - 1):
        wait_store(b)

    @pl.when(w < NW - 1)
    def _():
        wait_store(NBUF - 1)
        pltpu.make_async_copy(
            tbl_hbm.at[idx_v.at[pl.ds(0, TAIL)]], tbuf, tgsem
        ).wait()
        pltpu.async_copy(
            tbuf, out_hbm.at[pl.ds(r0 + NFULL * CHUNK, TAIL)], tssem
        )
        pltpu.make_async_copy(
            tbuf, out_hbm.at[pl.ds(0, TAIL)], tssem
        ).wait()

    @pl.when(w == NW - 1)
    def _():
        pltpu.make_async_copy(
            tbl_hbm.at[idx_v.at[pl.ds(0, TAIL31)]],
            tbuf.at[pl.ds(0, TAIL31)],
            tgsem,
        ).wait()
        pltpu.async_copy(
            tbuf.at[pl.ds(0, TAIL31)],
            out_hbm.at[pl.ds(r0 + 47 * CHUNK, TAIL31)],
            tssem,
        )
        pltpu.make_async_copy(
            tbuf.at[pl.ds(0, TAIL31)],
            out_hbm.at[pl.ds(0, TAIL31)],
            tssem,
        ).wait()


def kernel(x, edge_index, batch, emb_table):
    idx = jnp.squeeze(x, axis=1)
    return _gather_kernel(idx, emb_table)
